# baseline (speedup 1.0000x reference)
import functools

import jax
import jax.numpy as jnp
from jax import lax
from jax.experimental import pallas as pl
from jax.experimental.pallas import tpu as pltpu
from jax.experimental.pallas import tpu_sc as plsc

N_NODES = 10000
N_EDGES = 320000
D = 128
D_PK = D // 2
NW = 32
EDGES_PER_TILE = N_EDGES // NW
CHUNK = 80
N_CHUNKS = EDGES_PER_TILE // CHUNK
NBUF = 8
L = 16


def _edge_kernel(z_hbm, src_hbm, dst_hbm, out_hbm,
                 idx_s, idx_d, out_v, rows_s, rows_d, sems_s, sems_d):
    sid = lax.axis_index("s")
    wid = sid * 2 + lax.axis_index("c")
    tile_base = wid * EDGES_PER_TILE
    lanes = lax.iota(jnp.int32, L)

    pltpu.sync_copy(src_hbm.at[pl.ds(tile_base, EDGES_PER_TILE)], idx_s)
    pltpu.sync_copy(dst_hbm.at[pl.ds(tile_base, EDGES_PER_TILE)], idx_d)

    def fire(cidx, b):
        off = cidx * CHUNK
        pltpu.async_copy(z_hbm.at[idx_s.at[pl.ds(off, CHUNK)]], rows_s[b],
                         sems_s[b])
        pltpu.async_copy(z_hbm.at[idx_d.at[pl.ds(off, CHUNK)]], rows_d[b],
                         sems_d[b])

    def drain(cidx, b):
        off = cidx * CHUNK
        pltpu.make_async_copy(z_hbm.at[idx_s.at[pl.ds(off, CHUNK)]],
                              rows_s[b], sems_s[b]).wait()
        pltpu.make_async_copy(z_hbm.at[idx_d.at[pl.ds(off, CHUNK)]],
                              rows_d[b], sems_d[b]).wait()

    def compute(cidx, b):
        rs, rd = rows_s[b], rows_d[b]

        def group_body(g, _):
            erow = lanes + g * L

            def col_body(k, carry):
                acc0, acc1 = carry
                col = jnp.full((L,), 1, jnp.int32) * k
                ai = plsc.load_gather(rs, [erow, col])
                bi = plsc.load_gather(rd, [erow, col])
                av = plsc.bitcast(ai, jnp.bfloat16)
                bv = plsc.bitcast(bi, jnp.bfloat16)
                p0, p1 = plsc.unpack(av * bv,
                                     format=plsc.PackFormat.INTERLEAVED,
                                     preferred_element_type=jnp.float32)
                return acc0 + p0, acc1 + p1

            acc0, acc1 = lax.fori_loop(
                0, D_PK, col_body,
                (jnp.zeros((L,), jnp.float32), jnp.zeros((L,), jnp.float32)),
                unroll=16)
            acc = acc0 + acc1
            y = 1.0 / (1.0 + jnp.exp(-acc))
            out_v[pl.ds(cidx * CHUNK + g * L, L)] = y
            return _

        lax.fori_loop(0, CHUNK // L, group_body, None)

    for c in range(NBUF - 1):
        fire(c, c)

    def outer_body(c4, _):
        for b in range(NBUF):
            cidx = c4 * NBUF + b
            nxt = cidx + (NBUF - 1)

            @pl.when(nxt < N_CHUNKS)
            def _():
                fire(nxt, (b + NBUF - 1) % NBUF)

            drain(cidx, b)
            compute(cidx, b)
        return _

    M = ((N_CHUNKS - 1) // NBUF) * NBUF
    lax.fori_loop(0, M // NBUF, outer_body, None)
    for cidx in range(M, N_CHUNKS):
        b = cidx % NBUF
        nxt = cidx + (NBUF - 1)
        if nxt < N_CHUNKS:
            fire(nxt, nxt % NBUF)
        drain(cidx, b)
        compute(cidx, b)

    pltpu.sync_copy(out_v, out_hbm.at[pl.ds(tile_base, EDGES_PER_TILE)])


@jax.jit
def _decode(z, src, dst):
    z_pk = jax.lax.bitcast_convert_type(
        z.astype(jnp.bfloat16).reshape(N_NODES, D_PK, 2), jnp.int32)
    mesh = plsc.VectorSubcoreMesh(core_axis_name="c", subcore_axis_name="s")
    fn = functools.partial(
        pl.kernel,
        mesh=mesh,
        out_type=jax.ShapeDtypeStruct((N_EDGES,), jnp.float32),
        compiler_params=pltpu.CompilerParams(needs_layout_passes=False,
                                             use_tc_tiling_on_sc=False),
        scratch_types=[
            pltpu.VMEM((EDGES_PER_TILE,), jnp.int32),
            pltpu.VMEM((EDGES_PER_TILE,), jnp.int32),
            pltpu.VMEM((EDGES_PER_TILE,), jnp.float32),
            [pltpu.VMEM((CHUNK, D_PK), jnp.int32) for _ in range(NBUF)],
            [pltpu.VMEM((CHUNK, D_PK), jnp.int32) for _ in range(NBUF)],
            [pltpu.SemaphoreType.DMA for _ in range(NBUF)],
            [pltpu.SemaphoreType.DMA for _ in range(NBUF)],
        ],
    )(_edge_kernel)
    return fn(z_pk, src, dst)


def kernel(z, edge_index):
    return _decode(z, edge_index[0], edge_index[1])


# speedup vs baseline: 1.0058x; 1.0058x over previous
import functools

import jax
import jax.numpy as jnp
from jax import lax
from jax.experimental import pallas as pl
from jax.experimental.pallas import tpu as pltpu
from jax.experimental.pallas import tpu_sc as plsc

N_NODES = 10000
N_EDGES = 320000
D = 128
D_PK = D // 2
NW = 32
EDGES_PER_TILE = N_EDGES // NW
CHUNK = 80
N_CHUNKS = EDGES_PER_TILE // CHUNK
NBUF = 6
L = 16


def _edge_kernel(z_hbm, src_hbm, dst_hbm, out_hbm,
                 idx_s, idx_d, out_v, rows_s, rows_d, sems_s, sems_d):
    sid = lax.axis_index("s")
    wid = sid * 2 + lax.axis_index("c")
    tile_base = wid * EDGES_PER_TILE
    lanes = lax.iota(jnp.int32, L)

    pltpu.sync_copy(src_hbm.at[pl.ds(tile_base, EDGES_PER_TILE)], idx_s)
    pltpu.sync_copy(dst_hbm.at[pl.ds(tile_base, EDGES_PER_TILE)], idx_d)

    def fire(cidx, b):
        off = cidx * CHUNK
        pltpu.async_copy(z_hbm.at[idx_s.at[pl.ds(off, CHUNK)]], rows_s[b],
                         sems_s[b])
        pltpu.async_copy(z_hbm.at[idx_d.at[pl.ds(off, CHUNK)]], rows_d[b],
                         sems_d[b])

    def drain(cidx, b):
        off = cidx * CHUNK
        pltpu.make_async_copy(z_hbm.at[idx_s.at[pl.ds(off, CHUNK)]],
                              rows_s[b], sems_s[b]).wait()
        pltpu.make_async_copy(z_hbm.at[idx_d.at[pl.ds(off, CHUNK)]],
                              rows_d[b], sems_d[b]).wait()

    def compute(cidx, b):
        rs, rd = rows_s[b], rows_d[b]

        def group_body(g, _):
            erow = lanes + g * L

            def col_body(k, carry):
                acc0, acc1 = carry
                col = jnp.full((L,), 1, jnp.int32) * k
                ai = plsc.load_gather(rs, [erow, col])
                bi = plsc.load_gather(rd, [erow, col])
                av = plsc.bitcast(ai, jnp.bfloat16)
                bv = plsc.bitcast(bi, jnp.bfloat16)
                p0, p1 = plsc.unpack(av * bv,
                                     format=plsc.PackFormat.INTERLEAVED,
                                     preferred_element_type=jnp.float32)
                return acc0 + p0, acc1 + p1

            acc0, acc1 = lax.fori_loop(
                0, D_PK, col_body,
                (jnp.zeros((L,), jnp.float32), jnp.zeros((L,), jnp.float32)),
                unroll=8)
            acc = acc0 + acc1
            y = 1.0 / (1.0 + jnp.exp(-acc))
            out_v[pl.ds(cidx * CHUNK + g * L, L)] = y
            return _

        lax.fori_loop(0, CHUNK // L, group_body, None)

    for c in range(NBUF - 1):
        fire(c, c)

    def outer_body(c4, _):
        for b in range(NBUF):
            cidx = c4 * NBUF + b
            nxt = cidx + (NBUF - 1)

            @pl.when(nxt < N_CHUNKS)
            def _():
                fire(nxt, (b + NBUF - 1) % NBUF)

            drain(cidx, b)
            compute(cidx, b)
        return _

    M = ((N_CHUNKS - 1) // NBUF) * NBUF
    lax.fori_loop(0, M // NBUF, outer_body, None)
    for cidx in range(M, N_CHUNKS):
        b = cidx % NBUF
        nxt = cidx + (NBUF - 1)
        if nxt < N_CHUNKS:
            fire(nxt, nxt % NBUF)
        drain(cidx, b)
        compute(cidx, b)

    pltpu.sync_copy(out_v, out_hbm.at[pl.ds(tile_base, EDGES_PER_TILE)])


@jax.jit
def _decode(z, src, dst):
    z_pk = jax.lax.bitcast_convert_type(
        z.astype(jnp.bfloat16).reshape(N_NODES, D_PK, 2), jnp.int32)
    mesh = plsc.VectorSubcoreMesh(core_axis_name="c", subcore_axis_name="s")
    fn = functools.partial(
        pl.kernel,
        mesh=mesh,
        out_type=jax.ShapeDtypeStruct((N_EDGES,), jnp.float32),
        compiler_params=pltpu.CompilerParams(needs_layout_passes=False,
                                             use_tc_tiling_on_sc=False),
        scratch_types=[
            pltpu.VMEM((EDGES_PER_TILE,), jnp.int32),
            pltpu.VMEM((EDGES_PER_TILE,), jnp.int32),
            pltpu.VMEM((EDGES_PER_TILE,), jnp.float32),
            [pltpu.VMEM((CHUNK, D_PK), jnp.int32) for _ in range(NBUF)],
            [pltpu.VMEM((CHUNK, D_PK), jnp.int32) for _ in range(NBUF)],
            [pltpu.SemaphoreType.DMA for _ in range(NBUF)],
            [pltpu.SemaphoreType.DMA for _ in range(NBUF)],
        ],
    )(_edge_kernel)
    return fn(z_pk, src, dst)


def kernel(z, edge_index):
    return _decode(z, edge_index[0], edge_index[1])


# speedup vs baseline: 1.0086x; 1.0028x over previous
import functools

import jax
import jax.numpy as jnp
from jax import lax
from jax.experimental import pallas as pl
from jax.experimental.pallas import tpu as pltpu
from jax.experimental.pallas import tpu_sc as plsc

N_NODES = 10000
N_EDGES = 320000
D = 128
D_PK = D // 2
NW = 32
EDGES_PER_TILE = N_EDGES // NW
CHUNK = 80
N_CHUNKS = EDGES_PER_TILE // CHUNK
NBUF = 6
L = 16


def _edge_kernel(z_hbm, src_hbm, dst_hbm, out_hbm,
                 idx_s, idx_d, out_v, rows_s, rows_d, sems_s, sems_d, sem_o):
    sid = lax.axis_index("s")
    wid = sid * 2 + lax.axis_index("c")
    tile_base = wid * EDGES_PER_TILE
    lanes = lax.iota(jnp.int32, L)

    cp_is = pltpu.async_copy(src_hbm.at[pl.ds(tile_base, EDGES_PER_TILE)],
                             idx_s, sems_s[0])
    cp_id = pltpu.async_copy(dst_hbm.at[pl.ds(tile_base, EDGES_PER_TILE)],
                             idx_d, sems_d[0])
    cp_is.wait()
    cp_id.wait()

    def fire(cidx, b):
        off = cidx * CHUNK
        pltpu.async_copy(z_hbm.at[idx_s.at[pl.ds(off, CHUNK)]], rows_s[b],
                         sems_s[b])
        pltpu.async_copy(z_hbm.at[idx_d.at[pl.ds(off, CHUNK)]], rows_d[b],
                         sems_d[b])

    def drain(cidx, b):
        off = cidx * CHUNK
        pltpu.make_async_copy(z_hbm.at[idx_s.at[pl.ds(off, CHUNK)]],
                              rows_s[b], sems_s[b]).wait()
        pltpu.make_async_copy(z_hbm.at[idx_d.at[pl.ds(off, CHUNK)]],
                              rows_d[b], sems_d[b]).wait()

    def compute(cidx, b):
        rs, rd = rows_s[b], rows_d[b]

        def group_body(g, _):
            erow = lanes + g * L

            def col_body(k, carry):
                acc0, acc1 = carry
                col = jnp.full((L,), 1, jnp.int32) * k
                ai = plsc.load_gather(rs, [erow, col])
                bi = plsc.load_gather(rd, [erow, col])
                av = plsc.bitcast(ai, jnp.bfloat16)
                bv = plsc.bitcast(bi, jnp.bfloat16)
                p0, p1 = plsc.unpack(av * bv,
                                     format=plsc.PackFormat.INTERLEAVED,
                                     preferred_element_type=jnp.float32)
                return acc0 + p0, acc1 + p1

            acc0, acc1 = lax.fori_loop(
                0, D_PK, col_body,
                (jnp.zeros((L,), jnp.float32), jnp.zeros((L,), jnp.float32)),
                unroll=8)
            acc = acc0 + acc1
            y = 1.0 / (1.0 + jnp.exp(-acc))
            out_v[pl.ds(cidx * CHUNK + g * L, L)] = y
            return _

        lax.fori_loop(0, CHUNK // L, group_body, None)

    for c in range(NBUF - 1):
        fire(c, c)

    def outer_body(c4, _):
        for b in range(NBUF):
            cidx = c4 * NBUF + b
            nxt = cidx + (NBUF - 1)

            @pl.when(nxt < N_CHUNKS)
            def _():
                fire(nxt, (b + NBUF - 1) % NBUF)

            drain(cidx, b)
            compute(cidx, b)
        return _

    M = ((N_CHUNKS - 1) // NBUF) * NBUF
    n_outer = M // NBUF
    half_e = (N_CHUNKS // 2) * CHUNK

    def outer_with_flush(c4, _):
        outer_body(c4, _)

        @pl.when(c4 == (N_CHUNKS // 2) // NBUF)
        def _():
            pltpu.async_copy(out_v.at[pl.ds(0, half_e)],
                             out_hbm.at[pl.ds(tile_base, half_e)], sem_o)

        return _

    lax.fori_loop(0, n_outer, outer_with_flush, None)
    for cidx in range(M, N_CHUNKS):
        b = cidx % NBUF
        nxt = cidx + (NBUF - 1)
        if nxt < N_CHUNKS:
            fire(nxt, nxt % NBUF)
        drain(cidx, b)
        compute(cidx, b)

    pltpu.sync_copy(out_v.at[pl.ds(half_e, EDGES_PER_TILE - half_e)],
                    out_hbm.at[pl.ds(tile_base + half_e,
                                     EDGES_PER_TILE - half_e)])
    pltpu.make_async_copy(out_v.at[pl.ds(0, half_e)],
                          out_hbm.at[pl.ds(tile_base, half_e)], sem_o).wait()


@jax.jit
def _decode(z, src, dst):
    z_pk = jax.lax.bitcast_convert_type(
        z.astype(jnp.bfloat16).reshape(N_NODES, D_PK, 2), jnp.int32)
    mesh = plsc.VectorSubcoreMesh(core_axis_name="c", subcore_axis_name="s")
    fn = functools.partial(
        pl.kernel,
        mesh=mesh,
        out_type=jax.ShapeDtypeStruct((N_EDGES,), jnp.float32),
        compiler_params=pltpu.CompilerParams(needs_layout_passes=False,
                                             use_tc_tiling_on_sc=False),
        scratch_types=[
            pltpu.VMEM((EDGES_PER_TILE,), jnp.int32),
            pltpu.VMEM((EDGES_PER_TILE,), jnp.int32),
            pltpu.VMEM((EDGES_PER_TILE,), jnp.float32),
            [pltpu.VMEM((CHUNK, D_PK), jnp.int32) for _ in range(NBUF)],
            [pltpu.VMEM((CHUNK, D_PK), jnp.int32) for _ in range(NBUF)],
            [pltpu.SemaphoreType.DMA for _ in range(NBUF)],
            [pltpu.SemaphoreType.DMA for _ in range(NBUF)],
            pltpu.SemaphoreType.DMA,
        ],
    )(_edge_kernel)
    return fn(z_pk, src, dst)


def kernel(z, edge_index):
    return _decode(z, edge_index[0], edge_index[1])
